# baseline (speedup 1.0000x reference)
import functools

import jax
import jax.numpy as jnp
from jax import lax
from jax.experimental import pallas as pl
from jax.experimental.pallas import tpu as pltpu
from jax.experimental.pallas import tpu_sc as plsc

BATCH = 16384
EMBED = 32
CHUNK = 128


def _sc_gather(user_ids, item_ids, user_table, item_table):
    info = plsc.get_sparse_core_info()
    nw = info.num_cores * info.num_subcores
    b_per_w = BATCH // nw
    nchunk = b_per_w // CHUNK
    mesh = plsc.VectorSubcoreMesh(core_axis_name="c", subcore_axis_name="s")

    @functools.partial(
        pl.kernel,
        mesh=mesh,
        out_type=[
            jax.ShapeDtypeStruct((BATCH, EMBED), jnp.float32),
            jax.ShapeDtypeStruct((BATCH, EMBED), jnp.float32),
        ],
        scratch_types=[
            pltpu.VMEM((b_per_w,), jnp.int32),
            pltpu.VMEM((b_per_w,), jnp.int32),
            pltpu.VMEM((b_per_w, EMBED), jnp.float32),
            pltpu.VMEM((b_per_w, EMBED), jnp.float32),
            pltpu.SemaphoreType.DMA,
        ],
        compiler_params=pltpu.CompilerParams(use_tc_tiling_on_sc=False),
    )
    def gather(uid_hbm, iid_hbm, ut_hbm, it_hbm, u_out, i_out,
               uidx, iidx, urows, irows, sem):
        wid = lax.axis_index("s") * info.num_cores + lax.axis_index("c")
        base = wid * b_per_w
        pltpu.sync_copy(uid_hbm.at[pl.ds(base, b_per_w)], uidx)
        pltpu.sync_copy(iid_hbm.at[pl.ds(base, b_per_w)], iidx)
        copies = []
        for j in range(nchunk):
            sl = pl.ds(j * CHUNK, CHUNK)
            copies.append(pltpu.async_copy(ut_hbm.at[uidx.at[sl]], urows.at[sl], sem))
            copies.append(pltpu.async_copy(it_hbm.at[iidx.at[sl]], irows.at[sl], sem))
        for c in copies:
            c.wait()
        pltpu.sync_copy(urows, u_out.at[pl.ds(base, b_per_w)])
        pltpu.sync_copy(irows, i_out.at[pl.ds(base, b_per_w)])

    return gather(user_ids, item_ids, user_table, item_table)


def _mlp_body(u, i, w1a, w1b, b1, w2, b2, w3, b3, wout, bout, o):
    h = u[...] @ w1a[...] + i[...] @ w1b[...] + b1[...]
    h = jnp.maximum(h, 0.0)
    h = jnp.maximum(h @ w2[...] + b2[...], 0.0)
    h = jnp.maximum(h @ w3[...] + b3[...], 0.0)
    z = jnp.sum(h * wout[...], axis=1, keepdims=True) + bout[...]
    o[...] = 1.0 / (1.0 + jnp.exp(-z))


def _tc_mlp(u_emb, i_emb, W1, b1, W2, b2, W3, b3, Wout, bout):
    blk = 2048
    grid = (BATCH // blk,)
    w1a, w1b = W1[:EMBED], W1[EMBED:]
    full = lambda shape: pl.BlockSpec(shape, lambda g: (0, 0))
    return pl.pallas_call(
        _mlp_body,
        grid=grid,
        in_specs=[
            pl.BlockSpec((blk, EMBED), lambda g: (g, 0)),
            pl.BlockSpec((blk, EMBED), lambda g: (g, 0)),
            full(w1a.shape), full(w1b.shape), full((1, 64)),
            full(W2.shape), full((1, 32)),
            full(W3.shape), full((1, 16)),
            full((1, 16)), full((1, 1)),
        ],
        out_specs=pl.BlockSpec((blk, 1), lambda g: (g, 0)),
        out_shape=jax.ShapeDtypeStruct((BATCH, 1), jnp.float32),
    )(u_emb, i_emb, w1a, w1b, b1.reshape(1, -1), W2, b2.reshape(1, -1),
      W3, b3.reshape(1, -1), Wout.reshape(1, -1), bout.reshape(1, 1))


def kernel(user_ids, item_ids, user_table, item_table,
           W1, b1, W2, b2, W3, b3, Wout, bout):
    u_emb, i_emb = _sc_gather(user_ids.astype(jnp.int32),
                              item_ids.astype(jnp.int32),
                              user_table, item_table)
    return _tc_mlp(u_emb, i_emb, W1, b1, W2, b2, W3, b3, Wout, bout)
